# baseline (speedup 1.0000x reference)
import functools

import jax
import jax.numpy as jnp
from jax import lax
from jax.experimental import pallas as pl
from jax.experimental.pallas import tpu as pltpu
from jax.experimental.pallas import tpu_sc as plsc

N = 10000
E = 320000
HID = 128
MSG = 128

NC = 2
NS = 16
NW = NC * NS
EPW = E // NW
CH = 80
SUB = 5
S = CH * SUB
NIT1 = EPW // S
NIT2 = EPW // CH
STRIPE = N // NS
ZR = STRIPE // 5
ZRS = 25

BE = 3200
BN = 1000

_P = jax.lax.Precision.HIGHEST
_f32 = jnp.float32


def _mesh():
    return plsc.VectorSubcoreMesh(
        core_axis_name="c", subcore_axis_name="s", num_cores=NC, num_subcores=NS)


def _dot(a, b):
    return jax.lax.dot(a, b, precision=_P, preferred_element_type=_f32)


def _dot16(a, b):
    return jax.lax.dot(a.astype(jnp.bfloat16), b.astype(jnp.bfloat16),
                       preferred_element_type=_f32)


def _pre_body(h, w1a, w1b, b1, t1, t2):
    hv = h[...]
    t1[...] = _dot(hv, w1a[...]) + b1[...]
    t2[...] = _dot(hv, w1b[...])


def _precompute(hidden, w1a, w1b, b1):
    grid = (N // BN,)
    return pl.pallas_call(
        _pre_body,
        grid=grid,
        in_specs=[
            pl.BlockSpec((BN, HID), lambda i: (i, 0)),
            pl.BlockSpec((HID, MSG), lambda i: (0, 0)),
            pl.BlockSpec((HID, MSG), lambda i: (0, 0)),
            pl.BlockSpec((1, MSG), lambda i: (0, 0)),
        ],
        out_specs=[
            pl.BlockSpec((BN, MSG), lambda i: (i, 0)),
            pl.BlockSpec((BN, MSG), lambda i: (i, 0)),
        ],
        out_shape=[
            jax.ShapeDtypeStruct((N, MSG), _f32),
            jax.ShapeDtypeStruct((N, MSG), _f32),
        ],
    )(hidden, w1a, w1b, b1)


SG = 200
GSUB = (80, 80, 40)
NSTEP = EPW // SG


def _sc_gather_wide(src, dst, t1, t2):
    @functools.partial(
        pl.kernel,
        out_type=(
            jax.ShapeDtypeStruct((E, MSG), _f32),
            jax.ShapeDtypeStruct((E, MSG), _f32),
        ),
        mesh=_mesh(),
        scratch_types=[
            pltpu.VMEM((SG,), jnp.int32),
            pltpu.VMEM((SG,), jnp.int32),
            pltpu.VMEM((SG,), jnp.int32),
            pltpu.VMEM((SG,), jnp.int32),
            pltpu.VMEM((SG, MSG), _f32),
            pltpu.VMEM((SG, MSG), _f32),
            pltpu.VMEM((SG, MSG), _f32),
            pltpu.VMEM((SG, MSG), _f32),
            pltpu.SemaphoreType.DMA,
            pltpu.SemaphoreType.DMA,
            pltpu.SemaphoreType.DMA,
            pltpu.SemaphoreType.DMA,
            pltpu.SemaphoreType.DMA,
            pltpu.SemaphoreType.DMA,
        ],
    )
    def gk(src_h, dst_h, t1_h, t2_h, g1_h, g2_h,
           idxs0, idxs1, idxd0, idxd1, ra0, ra1, rb0, rb1,
           semi0, semi1, semg0, semg1, semo0, semo1):
        wid = lax.axis_index("s") * NC + lax.axis_index("c")
        base = wid * EPW
        idxs = (idxs0, idxs1)
        idxd = (idxd0, idxd1)
        ra = (ra0, ra1)
        rb = (rb0, rb1)
        semi = (semi0, semi1)
        semg = (semg0, semg1)
        semo = (semo0, semo1)

        def fire_idx(j, b):
            off = base + j * SG
            pltpu.async_copy(src_h.at[pl.ds(off, SG)], idxs[b], semi[b])
            pltpu.async_copy(dst_h.at[pl.ds(off, SG)], idxd[b], semi[b])

        def drain_idx(b):
            pltpu.make_async_copy(src_h.at[pl.ds(0, SG)], idxs[b], semi[b]).wait()
            pltpu.make_async_copy(dst_h.at[pl.ds(0, SG)], idxd[b], semi[b]).wait()

        def fire_gathers(b):
            o = 0
            for c in GSUB:
                sl = pl.ds(o, c)
                pltpu.async_copy(t1_h.at[idxs[b].at[sl]], ra[b].at[sl], semg[b])
                pltpu.async_copy(t2_h.at[idxd[b].at[sl]], rb[b].at[sl], semg[b])
                o += c

        def drain_gathers(b):
            o = 0
            for c in GSUB:
                sl = pl.ds(o, c)
                pltpu.make_async_copy(
                    t1_h.at[idxs[b].at[sl]], ra[b].at[sl], semg[b]).wait()
                pltpu.make_async_copy(
                    t2_h.at[idxd[b].at[sl]], rb[b].at[sl], semg[b]).wait()
                o += c

        def fire_stores(j, b):
            off = base + j * SG
            pltpu.async_copy(ra[b], g1_h.at[pl.ds(off, SG)], semo[b])
            pltpu.async_copy(rb[b], g2_h.at[pl.ds(off, SG)], semo[b])

        def drain_stores(b):
            pltpu.make_async_copy(ra[b], g1_h.at[pl.ds(0, SG)], semo[b]).wait()
            pltpu.make_async_copy(rb[b], g2_h.at[pl.ds(0, SG)], semo[b]).wait()

        fire_idx(0, 0)
        fire_idx(1, 1)

        def body(jj, carry):
            for b in range(2):
                j = 2 * jj + b

                @pl.when(jj > 0)
                def _():
                    drain_stores(b)

                drain_idx(b)
                fire_gathers(b)
                drain_gathers(b)

                @pl.when(j + 2 < NSTEP)
                def _():
                    fire_idx(j + 2, b)

                fire_stores(j, b)
            return carry

        lax.fori_loop(0, NSTEP // 2, body, 0)
        drain_stores(0)
        drain_stores(1)

    return gk(src, dst, t1, t2)


def _sc_gather_coords(src, dst, c8):
    @functools.partial(
        pl.kernel,
        out_type=(
            jax.ShapeDtypeStruct((E, 8), _f32),
            jax.ShapeDtypeStruct((E, 8), _f32),
        ),
        mesh=_mesh(),
        scratch_types=[
            pltpu.VMEM((S,), jnp.int32),
            pltpu.VMEM((S,), jnp.int32),
            pltpu.VMEM((S, 8), _f32),
            pltpu.VMEM((S, 8), _f32),
            pltpu.SemaphoreType.DMA,
            pltpu.SemaphoreType.DMA,
        ],
        compiler_params=pltpu.CompilerParams(use_tc_tiling_on_sc=False),
    )
    def gk(src_h, dst_h, c8_h, c1_h, c2_h, idxs, idxd, ca, cb, sem1, sem2):
        wid = lax.axis_index("s") * NC + lax.axis_index("c")
        base = wid * EPW

        def body(j, carry):
            off = base + j * S
            cp1 = pltpu.async_copy(src_h.at[pl.ds(off, S)], idxs, sem1)
            cp2 = pltpu.async_copy(dst_h.at[pl.ds(off, S)], idxd, sem1)
            cp1.wait()
            cp2.wait()
            descs = []
            for k in range(SUB):
                sl = pl.ds(k * CH, CH)
                descs.append(pltpu.async_copy(c8_h.at[idxs.at[sl]], ca.at[sl], sem2))
                descs.append(pltpu.async_copy(c8_h.at[idxd.at[sl]], cb.at[sl], sem2))
            for d in descs:
                d.wait()
            pltpu.sync_copy(ca, c1_h.at[pl.ds(off, S)])
            pltpu.sync_copy(cb, c2_h.at[pl.ds(off, S)])
            return carry

        lax.fori_loop(0, NIT1, body, 0)

    return gk(src, dst, c8)


def _edge_body(g1, g2, c1, c2, wl2, wm2, bm2, wa, bav, wc1, bc1, wc2,
               m_out, ct_out, c1v, c2v, csem):
    i = pl.program_id(0)
    cp1 = pltpu.make_async_copy(c1.at[pl.ds(i * BE, BE)], c1v, csem)
    cp2 = pltpu.make_async_copy(c2.at[pl.ds(i * BE, BE)], c2v, csem)
    cp1.start()
    cp2.start()
    cp1.wait()
    cp2.wait()
    d = c1v[...] - c2v[...]
    l2 = jnp.sqrt(jnp.sum(d * d, axis=1, keepdims=True))
    pre = g1[...] + g2[...] + l2 * wl2[...]
    m1 = jnp.tanh(pre)
    m2 = jnp.tanh(_dot16(m1, wm2[...]) + bm2[...])
    att = jax.nn.sigmoid(
        jnp.sum(m2 * wa[...], axis=1, keepdims=True) + bav[...])
    mij = att * m2
    c = jnp.tanh(_dot16(mij, wc1[...]) + bc1[...])
    s = jnp.tanh(jnp.sum(c * wc2[...], axis=1, keepdims=True))
    m_out[...] = mij
    ct = d * s
    one = jnp.ones((ct.shape[0], 1), _f32)
    ct_out[...] = jnp.concatenate(
        [ct[:, :3], one, jnp.zeros((ct.shape[0], 12), _f32)], axis=1)


def _edge_mlp(g1, g2, c1, c2, wl2, wm2, bm2, wa, bav, wc1, bc1, wc2):
    grid = (E // BE,)
    full = lambda i: (0, 0)
    return pl.pallas_call(
        _edge_body,
        grid=grid,
        in_specs=[
            pl.BlockSpec((BE, MSG), lambda i: (i, 0)),
            pl.BlockSpec((BE, MSG), lambda i: (i, 0)),
            pl.BlockSpec(memory_space=pl.ANY),
            pl.BlockSpec(memory_space=pl.ANY),
            pl.BlockSpec((1, MSG), full),
            pl.BlockSpec((MSG, MSG), full),
            pl.BlockSpec((1, MSG), full),
            pl.BlockSpec((1, MSG), full),
            pl.BlockSpec((1, 1), full),
            pl.BlockSpec((MSG, MSG), full),
            pl.BlockSpec((1, MSG), full),
            pl.BlockSpec((1, MSG), full),
        ],
        out_specs=[
            pl.BlockSpec((BE, MSG), lambda i: (i, 0)),
            pl.BlockSpec((BE, 16), lambda i: (i, 0)),
        ],
        out_shape=[
            jax.ShapeDtypeStruct((E, MSG), _f32),
            jax.ShapeDtypeStruct((E, 16), _f32),
        ],
        scratch_shapes=[
            pltpu.VMEM((BE, 8), _f32),
            pltpu.VMEM((BE, 8), _f32),
            pltpu.SemaphoreType.DMA,
        ],
    )(g1, g2, c1, c2, wl2, wm2, bm2, wa, bav, wc1, bc1, wc2)


def _sc_scatter(m, ct, dst):
    @functools.partial(
        pl.kernel,
        out_type=(
            jax.ShapeDtypeStruct((NC * N, MSG), _f32),
            jax.ShapeDtypeStruct((NC * N, 16), _f32),
        ),
        mesh=_mesh(),
        scratch_types=[
            pltpu.VMEM((CH,), jnp.int32),
            pltpu.VMEM((CH,), jnp.int32),
            pltpu.VMEM((CH, MSG), _f32),
            pltpu.VMEM((CH, MSG), _f32),
            pltpu.VMEM((CH, 16), _f32),
            pltpu.VMEM((CH, 16), _f32),
            pltpu.VMEM((ZRS, MSG), _f32),
            pltpu.VMEM((ZRS, 16), _f32),
            pltpu.VMEM_SHARED((N, MSG), _f32),
            pltpu.VMEM_SHARED((N, 16), _f32),
            pltpu.SemaphoreType.DMA,
            pltpu.SemaphoreType.DMA,
        ],
        compiler_params=pltpu.CompilerParams(use_tc_tiling_on_sc=False),
    )
    def sk(m_h, ct_h, dst_h, pm_h, pct_h,
           idx0, idx1, rows0, rows1, crows0, crows1, zb, zbc, acc_m, acc_ct,
           seml0, seml1):
        cid = lax.axis_index("c")
        sid = lax.axis_index("s")
        zv = jnp.zeros((16,), _f32)
        idx = (idx0, idx1)
        rows = (rows0, rows1)
        crows = (crows0, crows1)
        seml = (seml0, seml1)
        sems = (seml0, seml1)
        base = (sid * NC + cid) * EPW

        def fire_loads(j, b):
            off = base + j * CH
            pltpu.async_copy(dst_h.at[pl.ds(off, CH)], idx[b], seml[b])
            pltpu.async_copy(m_h.at[pl.ds(off, CH)], rows[b], seml[b])
            pltpu.async_copy(ct_h.at[pl.ds(off, CH)], crows[b], seml[b])

        def drain_loads(b):
            pltpu.make_async_copy(dst_h.at[pl.ds(0, CH)], idx[b], seml[b]).wait()
            pltpu.make_async_copy(m_h.at[pl.ds(0, CH)], rows[b], seml[b]).wait()
            pltpu.make_async_copy(ct_h.at[pl.ds(0, CH)], crows[b], seml[b]).wait()

        def fire_scat(b):
            pltpu.async_copy(rows[b], acc_m.at[idx[b]], sems[b], add=True)
            pltpu.async_copy(crows[b], acc_ct.at[idx[b]], sems[b], add=True)

        def drain_scat(b):
            pltpu.make_async_copy(rows[b], acc_m.at[idx[b]], sems[b]).wait()
            pltpu.make_async_copy(crows[b], acc_ct.at[idx[b]], sems[b]).wait()

        fire_loads(0, 0)
        fire_loads(1, 1)

        def zrow(t, carry):
            zb[t // 8, pl.ds((t % 8) * 16, 16)] = zv
            return carry

        lax.fori_loop(0, ZRS * 8, zrow, 0)

        def zrow2(i, carry):
            zbc[i, :] = zv
            return carry

        lax.fori_loop(0, ZRS, zrow2, 0)

        def zcopy(t, carry):
            ds = pl.ds(sid * STRIPE + t * ZRS, ZRS)
            pltpu.sync_copy(zb, acc_m.at[ds])
            pltpu.sync_copy(zbc, acc_ct.at[ds])
            return carry

        lax.fori_loop(0, STRIPE // ZRS, zcopy, 0)
        plsc.subcore_barrier()

        def body(jj, carry):
            for b in range(2):
                j = 2 * jj + b
                drain_loads(b)
                fire_scat(b)
                drain_scat(b)

                @pl.when(j + 2 < NIT2)
                def _():
                    fire_loads(j + 2, b)
            return carry

        lax.fori_loop(0, NIT2 // 2, body, 0)
        drain_loads(0)
        fire_scat(0)
        drain_scat(0)
        plsc.subcore_barrier()

        out_ds = pl.ds(cid * N + sid * STRIPE, STRIPE)
        pltpu.sync_copy(acc_m.at[pl.ds(sid * STRIPE, STRIPE)], pm_h.at[out_ds])
        pltpu.sync_copy(acc_ct.at[pl.ds(sid * STRIPE, STRIPE)], pct_h.at[out_ds])

    return sk(m, ct, dst)


def _node_body(h, c8, pm0, pm1, pct0, pct1, wh1a, wh1b, b1, wh2, b2,
               ho, co):
    mi = pm0[...] + pm1[...]
    ctv = pct0[...] + pct1[...]
    cnt = ctv[:, 3:4]
    co[...] = c8[...] + ctv[:, :8] / cnt
    hv = h[...]
    hm = jnp.tanh(_dot(hv, wh1a[...]) + _dot(mi, wh1b[...]) + b1[...])
    ho[...] = hv + _dot(hm, wh2[...]) + b2[...]


def _node(hidden, c8, pm, pct, wh1a, wh1b, b1, wh2, b2):
    grid = (N // BN,)
    full = lambda i: (0, 0)
    nb = N // BN
    return pl.pallas_call(
        _node_body,
        grid=grid,
        in_specs=[
            pl.BlockSpec((BN, HID), lambda i: (i, 0)),
            pl.BlockSpec((BN, 8), lambda i: (i, 0)),
            pl.BlockSpec((BN, MSG), lambda i: (i, 0)),
            pl.BlockSpec((BN, MSG), lambda i: (i + nb, 0)),
            pl.BlockSpec((BN, 16), lambda i: (i, 0)),
            pl.BlockSpec((BN, 16), lambda i: (i + nb, 0)),
            pl.BlockSpec((HID, MSG), full),
            pl.BlockSpec((MSG, MSG), full),
            pl.BlockSpec((1, MSG), full),
            pl.BlockSpec((MSG, HID), full),
            pl.BlockSpec((1, HID), full),
        ],
        out_specs=[
            pl.BlockSpec((BN, HID), lambda i: (i, 0)),
            pl.BlockSpec((BN, 8), lambda i: (i, 0)),
        ],
        out_shape=[
            jax.ShapeDtypeStruct((N, HID), _f32),
            jax.ShapeDtypeStruct((N, 8), _f32),
        ],
    )(hidden, c8, pm, pm, pct, pct, wh1a, wh1b, b1, wh2, b2)


def kernel(coords, hidden, edges, Wm1, bm1, Wm2, bm2, Wc1, bc1, Wc2,
           Wa, ba, Wh1, bh1, Wh2, bh2):
    src = edges[0].astype(jnp.int32)
    dst = edges[1].astype(jnp.int32)
    c8 = jnp.pad(coords, ((0, 0), (0, 5)))

    w1a = Wm1[:HID]
    w1b = Wm1[HID:2 * HID]
    wl2 = Wm1[2 * HID:]
    t1, t2 = _precompute(hidden, w1a, w1b, bm1.reshape(1, MSG))

    g1, g2 = _sc_gather_wide(src, dst, t1, t2)
    ce1, ce2 = _sc_gather_coords(src, dst, c8)

    m, ct = _edge_mlp(
        g1, g2, ce1, ce2, wl2, Wm2, bm2.reshape(1, MSG),
        Wa.reshape(1, MSG), ba.reshape(1, 1), Wc1, bc1.reshape(1, MSG),
        Wc2.reshape(1, MSG))

    pm, pct = _sc_scatter(m, ct, dst)

    ho, co8 = _node(
        hidden, c8, pm, pct,
        Wh1[:HID], Wh1[HID:], bh1.reshape(1, MSG), Wh2,
        bh2.reshape(1, HID))

    return (co8[:, :3], ho)


# speedup vs baseline: 1.2133x; 1.2133x over previous
import functools

import jax
import jax.numpy as jnp
from jax import lax
from jax.experimental import pallas as pl
from jax.experimental.pallas import tpu as pltpu
from jax.experimental.pallas import tpu_sc as plsc

N = 10000
E = 320000
HID = 128
MSG = 128

NC = 2
NS = 16
NW = NC * NS
EPW = E // NW
CH = 80
SUB = 5
S = CH * SUB
NIT1 = EPW // S
NIT2 = EPW // CH
STRIPE = N // NS
ZR = STRIPE // 5
ZRS = 25

BE = 3200
BN = 1000

_P = jax.lax.Precision.HIGHEST
_f32 = jnp.float32


def _mesh():
    return plsc.VectorSubcoreMesh(
        core_axis_name="c", subcore_axis_name="s", num_cores=NC, num_subcores=NS)


def _dot(a, b):
    return jax.lax.dot(a, b, precision=_P, preferred_element_type=_f32)


def _dot16(a, b):
    return jax.lax.dot(a.astype(jnp.bfloat16), b.astype(jnp.bfloat16),
                       preferred_element_type=_f32)


def _pre_body(h, w1a, w1b, b1, t1, t2):
    hv = h[...]
    t1[...] = _dot(hv, w1a[...]) + b1[...]
    t2[...] = _dot(hv, w1b[...])


def _precompute(hidden, w1a, w1b, b1):
    grid = (N // BN,)
    return pl.pallas_call(
        _pre_body,
        grid=grid,
        in_specs=[
            pl.BlockSpec((BN, HID), lambda i: (i, 0)),
            pl.BlockSpec((HID, MSG), lambda i: (0, 0)),
            pl.BlockSpec((HID, MSG), lambda i: (0, 0)),
            pl.BlockSpec((1, MSG), lambda i: (0, 0)),
        ],
        out_specs=[
            pl.BlockSpec((BN, MSG), lambda i: (i, 0)),
            pl.BlockSpec((BN, MSG), lambda i: (i, 0)),
        ],
        out_shape=[
            jax.ShapeDtypeStruct((N, MSG), _f32),
            jax.ShapeDtypeStruct((N, MSG), _f32),
        ],
    )(hidden, w1a, w1b, b1)


SG = 200
GSUB = (80, 80, 40)
NSTEP = EPW // SG


def _sc_gather_wide(src, dst, t1, t2):
    @functools.partial(
        pl.kernel,
        out_type=(
            jax.ShapeDtypeStruct((E, MSG), _f32),
            jax.ShapeDtypeStruct((E, MSG), _f32),
        ),
        mesh=_mesh(),
        scratch_types=[
            pltpu.VMEM((SG,), jnp.int32),
            pltpu.VMEM((SG,), jnp.int32),
            pltpu.VMEM((SG,), jnp.int32),
            pltpu.VMEM((SG,), jnp.int32),
            pltpu.VMEM((SG, MSG), _f32),
            pltpu.VMEM((SG, MSG), _f32),
            pltpu.VMEM((SG, MSG), _f32),
            pltpu.VMEM((SG, MSG), _f32),
            pltpu.SemaphoreType.DMA,
            pltpu.SemaphoreType.DMA,
            pltpu.SemaphoreType.DMA,
            pltpu.SemaphoreType.DMA,
            pltpu.SemaphoreType.DMA,
            pltpu.SemaphoreType.DMA,
        ],
    )
    def gk(src_h, dst_h, t1_h, t2_h, g1_h, g2_h,
           idxs0, idxs1, idxd0, idxd1, ra0, ra1, rb0, rb1,
           semi0, semi1, semg0, semg1, semo0, semo1):
        wid = lax.axis_index("s") * NC + lax.axis_index("c")
        base = wid * EPW
        idxs = (idxs0, idxs1)
        idxd = (idxd0, idxd1)
        ra = (ra0, ra1)
        rb = (rb0, rb1)
        semi = (semi0, semi1)
        semg = (semg0, semg1)
        semo = (semo0, semo1)

        def fire_idx(j, b):
            off = base + j * SG
            pltpu.async_copy(src_h.at[pl.ds(off, SG)], idxs[b], semi[b])
            pltpu.async_copy(dst_h.at[pl.ds(off, SG)], idxd[b], semi[b])

        def drain_idx(b):
            pltpu.make_async_copy(src_h.at[pl.ds(0, SG)], idxs[b], semi[b]).wait()
            pltpu.make_async_copy(dst_h.at[pl.ds(0, SG)], idxd[b], semi[b]).wait()

        def fire_gathers(b):
            o = 0
            for c in GSUB:
                sl = pl.ds(o, c)
                pltpu.async_copy(t1_h.at[idxs[b].at[sl]], ra[b].at[sl], semg[b])
                pltpu.async_copy(t2_h.at[idxd[b].at[sl]], rb[b].at[sl], semg[b])
                o += c

        def drain_gathers(b):
            o = 0
            for c in GSUB:
                sl = pl.ds(o, c)
                pltpu.make_async_copy(
                    t1_h.at[idxs[b].at[sl]], ra[b].at[sl], semg[b]).wait()
                pltpu.make_async_copy(
                    t2_h.at[idxd[b].at[sl]], rb[b].at[sl], semg[b]).wait()
                o += c

        def fire_stores(j, b):
            off = base + j * SG
            pltpu.async_copy(ra[b], g1_h.at[pl.ds(off, SG)], semo[b])
            pltpu.async_copy(rb[b], g2_h.at[pl.ds(off, SG)], semo[b])

        def drain_stores(b):
            pltpu.make_async_copy(ra[b], g1_h.at[pl.ds(0, SG)], semo[b]).wait()
            pltpu.make_async_copy(rb[b], g2_h.at[pl.ds(0, SG)], semo[b]).wait()

        fire_idx(0, 0)
        fire_idx(1, 1)

        def body(jj, carry):
            for b in range(2):
                j = 2 * jj + b

                @pl.when(jj > 0)
                def _():
                    drain_stores(b)

                drain_idx(b)
                fire_gathers(b)
                drain_gathers(b)

                @pl.when(j + 2 < NSTEP)
                def _():
                    fire_idx(j + 2, b)

                fire_stores(j, b)
            return carry

        lax.fori_loop(0, NSTEP // 2, body, 0)
        drain_stores(0)
        drain_stores(1)

    return gk(src, dst, t1, t2)


def _sc_gather_coords(src, dst, c8):
    @functools.partial(
        pl.kernel,
        out_type=(
            jax.ShapeDtypeStruct((E, 8), _f32),
            jax.ShapeDtypeStruct((E, 8), _f32),
        ),
        mesh=_mesh(),
        scratch_types=[
            pltpu.VMEM((S,), jnp.int32),
            pltpu.VMEM((S,), jnp.int32),
            pltpu.VMEM((S, 8), _f32),
            pltpu.VMEM((S, 8), _f32),
            pltpu.SemaphoreType.DMA,
            pltpu.SemaphoreType.DMA,
        ],
        compiler_params=pltpu.CompilerParams(use_tc_tiling_on_sc=False),
    )
    def gk(src_h, dst_h, c8_h, c1_h, c2_h, idxs, idxd, ca, cb, sem1, sem2):
        wid = lax.axis_index("s") * NC + lax.axis_index("c")
        base = wid * EPW

        def body(j, carry):
            off = base + j * S
            cp1 = pltpu.async_copy(src_h.at[pl.ds(off, S)], idxs, sem1)
            cp2 = pltpu.async_copy(dst_h.at[pl.ds(off, S)], idxd, sem1)
            cp1.wait()
            cp2.wait()
            descs = []
            for k in range(SUB):
                sl = pl.ds(k * CH, CH)
                descs.append(pltpu.async_copy(c8_h.at[idxs.at[sl]], ca.at[sl], sem2))
                descs.append(pltpu.async_copy(c8_h.at[idxd.at[sl]], cb.at[sl], sem2))
            for d in descs:
                d.wait()
            pltpu.sync_copy(ca, c1_h.at[pl.ds(off, S)])
            pltpu.sync_copy(cb, c2_h.at[pl.ds(off, S)])
            return carry

        lax.fori_loop(0, NIT1, body, 0)

    return gk(src, dst, c8)


def _edge_body(g1, g2, c1, c2, wl2, wm2, bm2, wa, bav, wc1, bc1, wc2,
               m_out, ct_out):
    d = c1[...] - c2[...]
    l2 = jnp.sqrt(jnp.sum(d * d, axis=1, keepdims=True))
    pre = g1[...] + g2[...] + l2 * wl2[...]
    m1 = jnp.tanh(pre)
    m2 = jnp.tanh(_dot16(m1, wm2[...]) + bm2[...])
    att = jax.nn.sigmoid(
        jnp.sum(m2 * wa[...], axis=1, keepdims=True) + bav[...])
    mij = att * m2
    c = jnp.tanh(_dot16(mij, wc1[...]) + bc1[...])
    s = jnp.tanh(jnp.sum(c * wc2[...], axis=1, keepdims=True))
    m_out[...] = mij
    ct = d * s
    one = jnp.ones((ct.shape[0], 1), _f32)
    ct_out[...] = jnp.concatenate(
        [ct[:, :3], one, jnp.zeros((ct.shape[0], 12), _f32)], axis=1)


def _edge_mlp(g1, g2, c1, c2, wl2, wm2, bm2, wa, bav, wc1, bc1, wc2):
    grid = (E // BE,)
    full = lambda i: (0, 0)
    return pl.pallas_call(
        _edge_body,
        grid=grid,
        in_specs=[
            pl.BlockSpec((BE, MSG), lambda i: (i, 0)),
            pl.BlockSpec((BE, MSG), lambda i: (i, 0)),
            pl.BlockSpec((BE, 8), lambda i: (i, 0)),
            pl.BlockSpec((BE, 8), lambda i: (i, 0)),
            pl.BlockSpec((1, MSG), full),
            pl.BlockSpec((MSG, MSG), full),
            pl.BlockSpec((1, MSG), full),
            pl.BlockSpec((1, MSG), full),
            pl.BlockSpec((1, 1), full),
            pl.BlockSpec((MSG, MSG), full),
            pl.BlockSpec((1, MSG), full),
            pl.BlockSpec((1, MSG), full),
        ],
        out_specs=[
            pl.BlockSpec((BE, MSG), lambda i: (i, 0)),
            pl.BlockSpec((BE, 16), lambda i: (i, 0)),
        ],
        out_shape=[
            jax.ShapeDtypeStruct((E, MSG), _f32),
            jax.ShapeDtypeStruct((E, 16), _f32),
        ],
    )(g1, g2, c1, c2, wl2, wm2, bm2, wa, bav, wc1, bc1, wc2)


def _sc_scatter_m(m, dst):
    ACCN = 10240
    ASTR = ACCN // NS
    ZM = 40

    @functools.partial(
        pl.kernel,
        out_type=jax.ShapeDtypeStruct((NC * ACCN, MSG), _f32),
        mesh=_mesh(),
        scratch_types=[
            pltpu.VMEM((CH,), jnp.int32),
            pltpu.VMEM((CH,), jnp.int32),
            pltpu.VMEM((CH, MSG), _f32),
            pltpu.VMEM((CH, MSG), _f32),
            pltpu.VMEM((ZM, MSG), _f32),
            pltpu.VMEM_SHARED((ACCN, MSG), _f32),
            pltpu.SemaphoreType.DMA,
            pltpu.SemaphoreType.DMA,
        ],
    )
    def sk(m_h, dst_h, pm_h, idx0, idx1, rows0, rows1, zb, acc_m, seml0, seml1):
        cid = lax.axis_index("c")
        sid = lax.axis_index("s")
        zv = jnp.zeros((16,), _f32)
        idx = (idx0, idx1)
        rows = (rows0, rows1)
        seml = (seml0, seml1)
        base = (sid * NC + cid) * EPW

        def fire_loads(j, b):
            off = base + j * CH
            pltpu.async_copy(dst_h.at[pl.ds(off, CH)], idx[b], seml[b])
            pltpu.async_copy(m_h.at[pl.ds(off, CH)], rows[b], seml[b])

        def drain_loads(b):
            pltpu.make_async_copy(dst_h.at[pl.ds(0, CH)], idx[b], seml[b]).wait()
            pltpu.make_async_copy(m_h.at[pl.ds(0, CH)], rows[b], seml[b]).wait()

        def fire_scat(b):
            pltpu.async_copy(rows[b], acc_m.at[idx[b]], seml[b], add=True)

        def drain_scat(b):
            pltpu.make_async_copy(rows[b], acc_m.at[idx[b]], seml[b]).wait()

        fire_loads(0, 0)
        fire_loads(1, 1)

        def zrow(t, carry):
            zb[t // 8, pl.ds((t % 8) * 16, 16)] = zv
            return carry

        lax.fori_loop(0, ZM * 8, zrow, 0)

        def zcopy(t, carry):
            pltpu.sync_copy(zb, acc_m.at[pl.ds(sid * ASTR + t * ZM, ZM)])
            return carry

        lax.fori_loop(0, ASTR // ZM, zcopy, 0)
        plsc.subcore_barrier()

        def body(jj, carry):
            for b in range(2):
                j = 2 * jj + b
                drain_loads(b)
                fire_scat(b)
                drain_scat(b)

                @pl.when(j + 2 < NIT2)
                def _():
                    fire_loads(j + 2, b)
            return carry

        lax.fori_loop(0, NIT2 // 2, body, 0)
        drain_loads(0)
        fire_scat(0)
        drain_scat(0)
        plsc.subcore_barrier()

        out_ds = pl.ds(cid * ACCN + sid * ASTR, ASTR)
        pltpu.sync_copy(acc_m.at[pl.ds(sid * ASTR, ASTR)], pm_h.at[out_ds])

    return sk(m, dst), ACCN


def _sc_scatter_ct(ct, dst):
    @functools.partial(
        pl.kernel,
        out_type=jax.ShapeDtypeStruct((NC * N, 16), _f32),
        mesh=_mesh(),
        scratch_types=[
            pltpu.VMEM((CH,), jnp.int32),
            pltpu.VMEM((CH,), jnp.int32),
            pltpu.VMEM((CH, 16), _f32),
            pltpu.VMEM((CH, 16), _f32),
            pltpu.VMEM((ZRS, 16), _f32),
            pltpu.VMEM_SHARED((N, 16), _f32),
            pltpu.SemaphoreType.DMA,
            pltpu.SemaphoreType.DMA,
        ],
        compiler_params=pltpu.CompilerParams(use_tc_tiling_on_sc=False),
    )
    def sk(ct_h, dst_h, pct_h, idx0, idx1, crows0, crows1, zbc, acc_ct,
           seml0, seml1):
        cid = lax.axis_index("c")
        sid = lax.axis_index("s")
        zv = jnp.zeros((16,), _f32)
        idx = (idx0, idx1)
        crows = (crows0, crows1)
        seml = (seml0, seml1)
        base = (sid * NC + cid) * EPW

        def fire_loads(j, b):
            off = base + j * CH
            pltpu.async_copy(dst_h.at[pl.ds(off, CH)], idx[b], seml[b])
            pltpu.async_copy(ct_h.at[pl.ds(off, CH)], crows[b], seml[b])

        def drain_loads(b):
            pltpu.make_async_copy(dst_h.at[pl.ds(0, CH)], idx[b], seml[b]).wait()
            pltpu.make_async_copy(ct_h.at[pl.ds(0, CH)], crows[b], seml[b]).wait()

        def fire_scat(b):
            pltpu.async_copy(crows[b], acc_ct.at[idx[b]], seml[b], add=True)

        def drain_scat(b):
            pltpu.make_async_copy(crows[b], acc_ct.at[idx[b]], seml[b]).wait()

        fire_loads(0, 0)
        fire_loads(1, 1)

        def zrow2(i, carry):
            zbc[i, :] = zv
            return carry

        lax.fori_loop(0, ZRS, zrow2, 0)

        def zcopy(t, carry):
            pltpu.sync_copy(zbc, acc_ct.at[pl.ds(sid * STRIPE + t * ZRS, ZRS)])
            return carry

        lax.fori_loop(0, STRIPE // ZRS, zcopy, 0)
        plsc.subcore_barrier()

        def body(jj, carry):
            for b in range(2):
                j = 2 * jj + b
                drain_loads(b)
                fire_scat(b)
                drain_scat(b)

                @pl.when(j + 2 < NIT2)
                def _():
                    fire_loads(j + 2, b)
            return carry

        lax.fori_loop(0, NIT2 // 2, body, 0)
        drain_loads(0)
        fire_scat(0)
        drain_scat(0)
        plsc.subcore_barrier()

        out_ds = pl.ds(cid * N + sid * STRIPE, STRIPE)
        pltpu.sync_copy(acc_ct.at[pl.ds(sid * STRIPE, STRIPE)], pct_h.at[out_ds])

    return sk(ct, dst)


def _node_body(h, c8, pm0, pm1, pct0, pct1, wh1a, wh1b, b1, wh2, b2,
               ho, co):
    mi = pm0[...] + pm1[...]
    ctv = pct0[...] + pct1[...]
    cnt = ctv[:, 3:4]
    co[...] = c8[...] + ctv[:, :8] / cnt
    hv = h[...]
    hm = jnp.tanh(_dot(hv, wh1a[...]) + _dot(mi, wh1b[...]) + b1[...])
    ho[...] = hv + _dot(hm, wh2[...]) + b2[...]


def _node(hidden, c8, pm, pct, wh1a, wh1b, b1, wh2, b2):
    grid = (N // BN,)
    full = lambda i: (0, 0)
    nb = N // BN
    return pl.pallas_call(
        _node_body,
        grid=grid,
        in_specs=[
            pl.BlockSpec((BN, HID), lambda i: (i, 0)),
            pl.BlockSpec((BN, 8), lambda i: (i, 0)),
            pl.BlockSpec((BN, MSG), lambda i: (i, 0)),
            pl.BlockSpec((BN, MSG), lambda i: (i + nb, 0)),
            pl.BlockSpec((BN, 16), lambda i: (i, 0)),
            pl.BlockSpec((BN, 16), lambda i: (i + nb, 0)),
            pl.BlockSpec((HID, MSG), full),
            pl.BlockSpec((MSG, MSG), full),
            pl.BlockSpec((1, MSG), full),
            pl.BlockSpec((MSG, HID), full),
            pl.BlockSpec((1, HID), full),
        ],
        out_specs=[
            pl.BlockSpec((BN, HID), lambda i: (i, 0)),
            pl.BlockSpec((BN, 8), lambda i: (i, 0)),
        ],
        out_shape=[
            jax.ShapeDtypeStruct((N, HID), _f32),
            jax.ShapeDtypeStruct((N, 8), _f32),
        ],
    )(hidden, c8, pm, pm, pct, pct, wh1a, wh1b, b1, wh2, b2)


def kernel(coords, hidden, edges, Wm1, bm1, Wm2, bm2, Wc1, bc1, Wc2,
           Wa, ba, Wh1, bh1, Wh2, bh2):
    src = edges[0].astype(jnp.int32)
    dst = edges[1].astype(jnp.int32)
    c8 = jnp.pad(coords, ((0, 0), (0, 5)))

    w1a = Wm1[:HID]
    w1b = Wm1[HID:2 * HID]
    wl2 = Wm1[2 * HID:]
    t1, t2 = _precompute(hidden, w1a, w1b, bm1.reshape(1, MSG))

    g1, g2 = _sc_gather_wide(src, dst, t1, t2)
    ce1, ce2 = _sc_gather_coords(src, dst, c8)

    m, ct = _edge_mlp(
        g1, g2, ce1, ce2, wl2, Wm2, bm2.reshape(1, MSG),
        Wa.reshape(1, MSG), ba.reshape(1, 1), Wc1, bc1.reshape(1, MSG),
        Wc2.reshape(1, MSG))

    pm_full, accn = _sc_scatter_m(m, dst)
    pct = _sc_scatter_ct(ct, dst)
    pm = jnp.concatenate([pm_full[:N], pm_full[accn:accn + N]], axis=0)

    ho, co8 = _node(
        hidden, c8, pm, pct,
        Wh1[:HID], Wh1[HID:], bh1.reshape(1, MSG), Wh2,
        bh2.reshape(1, HID))

    return (co8[:, :3], ho)


# speedup vs baseline: 1.2840x; 1.0583x over previous
import functools

import jax
import jax.numpy as jnp
from jax import lax
from jax.experimental import pallas as pl
from jax.experimental.pallas import tpu as pltpu
from jax.experimental.pallas import tpu_sc as plsc

N = 10000
E = 320000
HID = 128
MSG = 128

NC = 2
NS = 16
NW = NC * NS
EPW = E // NW
CH = 80
SUB = 5
S = CH * SUB
NIT1 = EPW // S
NIT2 = EPW // CH
STRIPE = N // NS
ZR = STRIPE // 5
ZRS = 25

BE = 6400
BN = 1000

_P = jax.lax.Precision.HIGHEST
_f32 = jnp.float32


def _mesh():
    return plsc.VectorSubcoreMesh(
        core_axis_name="c", subcore_axis_name="s", num_cores=NC, num_subcores=NS)


def _dot(a, b):
    return jax.lax.dot(a, b, precision=_P, preferred_element_type=_f32)


def _dot16(a, b):
    return jax.lax.dot(a.astype(jnp.bfloat16), b.astype(jnp.bfloat16),
                       preferred_element_type=_f32)


def _pre_body(h, w1a, w1b, b1, t1, t2):
    hv = h[...]
    t1[...] = _dot(hv, w1a[...]) + b1[...]
    t2[...] = _dot(hv, w1b[...])


def _precompute(hidden, w1a, w1b, b1):
    grid = (N // BN,)
    return pl.pallas_call(
        _pre_body,
        grid=grid,
        in_specs=[
            pl.BlockSpec((BN, HID), lambda i: (i, 0)),
            pl.BlockSpec((HID, MSG), lambda i: (0, 0)),
            pl.BlockSpec((HID, MSG), lambda i: (0, 0)),
            pl.BlockSpec((1, MSG), lambda i: (0, 0)),
        ],
        out_specs=[
            pl.BlockSpec((BN, MSG), lambda i: (i, 0)),
            pl.BlockSpec((BN, MSG), lambda i: (i, 0)),
        ],
        out_shape=[
            jax.ShapeDtypeStruct((N, MSG), _f32),
            jax.ShapeDtypeStruct((N, MSG), _f32),
        ],
    )(hidden, w1a, w1b, b1)


SG = 200
GSUB = (128, 72)
NSTEP = EPW // SG


def _sc_gather_wide(src, dst, t1, t2):
    @functools.partial(
        pl.kernel,
        out_type=(
            jax.ShapeDtypeStruct((E, MSG), _f32),
            jax.ShapeDtypeStruct((E, MSG), _f32),
        ),
        mesh=_mesh(),
        scratch_types=[
            pltpu.VMEM((SG,), jnp.int32),
            pltpu.VMEM((SG,), jnp.int32),
            pltpu.VMEM((SG,), jnp.int32),
            pltpu.VMEM((SG,), jnp.int32),
            pltpu.VMEM((SG, MSG), _f32),
            pltpu.VMEM((SG, MSG), _f32),
            pltpu.VMEM((SG, MSG), _f32),
            pltpu.VMEM((SG, MSG), _f32),
            pltpu.SemaphoreType.DMA,
            pltpu.SemaphoreType.DMA,
            pltpu.SemaphoreType.DMA,
            pltpu.SemaphoreType.DMA,
            pltpu.SemaphoreType.DMA,
            pltpu.SemaphoreType.DMA,
        ],
    )
    def gk(src_h, dst_h, t1_h, t2_h, g1_h, g2_h,
           idxs0, idxs1, idxd0, idxd1, ra0, ra1, rb0, rb1,
           semi0, semi1, semg0, semg1, semo0, semo1):
        wid = lax.axis_index("s") * NC + lax.axis_index("c")
        base = wid * EPW
        idxs = (idxs0, idxs1)
        idxd = (idxd0, idxd1)
        ra = (ra0, ra1)
        rb = (rb0, rb1)
        semi = (semi0, semi1)
        semg = (semg0, semg1)
        semo = (semo0, semo1)

        def fire_idx(j, b):
            off = base + j * SG
            pltpu.async_copy(src_h.at[pl.ds(off, SG)], idxs[b], semi[b])
            pltpu.async_copy(dst_h.at[pl.ds(off, SG)], idxd[b], semi[b])

        def drain_idx(b):
            pltpu.make_async_copy(src_h.at[pl.ds(0, SG)], idxs[b], semi[b]).wait()
            pltpu.make_async_copy(dst_h.at[pl.ds(0, SG)], idxd[b], semi[b]).wait()

        def fire_gathers(b):
            o = 0
            for c in GSUB:
                sl = pl.ds(o, c)
                pltpu.async_copy(t1_h.at[idxs[b].at[sl]], ra[b].at[sl], semg[b])
                pltpu.async_copy(t2_h.at[idxd[b].at[sl]], rb[b].at[sl], semg[b])
                o += c

        def drain_gathers(b):
            o = 0
            for c in GSUB:
                sl = pl.ds(o, c)
                pltpu.make_async_copy(
                    t1_h.at[idxs[b].at[sl]], ra[b].at[sl], semg[b]).wait()
                pltpu.make_async_copy(
                    t2_h.at[idxd[b].at[sl]], rb[b].at[sl], semg[b]).wait()
                o += c

        def fire_stores(j, b):
            off = base + j * SG
            pltpu.async_copy(ra[b], g1_h.at[pl.ds(off, SG)], semo[b])
            pltpu.async_copy(rb[b], g2_h.at[pl.ds(off, SG)], semo[b])

        def drain_stores(b):
            pltpu.make_async_copy(ra[b], g1_h.at[pl.ds(0, SG)], semo[b]).wait()
            pltpu.make_async_copy(rb[b], g2_h.at[pl.ds(0, SG)], semo[b]).wait()

        fire_idx(0, 0)
        fire_idx(1, 1)

        def body(jj, carry):
            for b in range(2):
                j = 2 * jj + b

                @pl.when(jj > 0)
                def _():
                    drain_stores(b)

                drain_idx(b)
                fire_gathers(b)
                drain_gathers(b)

                @pl.when(j + 2 < NSTEP)
                def _():
                    fire_idx(j + 2, b)

                fire_stores(j, b)
            return carry

        lax.fori_loop(0, NSTEP // 2, body, 0)
        drain_stores(0)
        drain_stores(1)

    return gk(src, dst, t1, t2)


def _sc_gather_coords(src, dst, c8):
    @functools.partial(
        pl.kernel,
        out_type=(
            jax.ShapeDtypeStruct((E, 8), _f32),
            jax.ShapeDtypeStruct((E, 8), _f32),
        ),
        mesh=_mesh(),
        scratch_types=[
            pltpu.VMEM((S,), jnp.int32),
            pltpu.VMEM((S,), jnp.int32),
            pltpu.VMEM((S, 8), _f32),
            pltpu.VMEM((S, 8), _f32),
            pltpu.SemaphoreType.DMA,
            pltpu.SemaphoreType.DMA,
        ],
        compiler_params=pltpu.CompilerParams(use_tc_tiling_on_sc=False),
    )
    def gk(src_h, dst_h, c8_h, c1_h, c2_h, idxs, idxd, ca, cb, sem1, sem2):
        wid = lax.axis_index("s") * NC + lax.axis_index("c")
        base = wid * EPW

        def body(j, carry):
            off = base + j * S
            cp1 = pltpu.async_copy(src_h.at[pl.ds(off, S)], idxs, sem1)
            cp2 = pltpu.async_copy(dst_h.at[pl.ds(off, S)], idxd, sem1)
            cp1.wait()
            cp2.wait()
            descs = []
            for k in range(SUB):
                sl = pl.ds(k * CH, CH)
                descs.append(pltpu.async_copy(c8_h.at[idxs.at[sl]], ca.at[sl], sem2))
                descs.append(pltpu.async_copy(c8_h.at[idxd.at[sl]], cb.at[sl], sem2))
            for d in descs:
                d.wait()
            pltpu.sync_copy(ca, c1_h.at[pl.ds(off, S)])
            pltpu.sync_copy(cb, c2_h.at[pl.ds(off, S)])
            return carry

        lax.fori_loop(0, NIT1, body, 0)

    return gk(src, dst, c8)


def _edge_body(g1, g2, c1, c2, wl2, wm2, bm2, wa, bav, wc1, bc1, wc2,
               m_out, ct_out):
    d = c1[...] - c2[...]
    l2 = jnp.sqrt(jnp.sum(d * d, axis=1, keepdims=True))
    pre = g1[...] + g2[...] + l2 * wl2[...]
    m1 = jnp.tanh(pre)
    m2 = jnp.tanh(_dot16(m1, wm2[...]) + bm2[...])
    att = jax.nn.sigmoid(
        jnp.sum(m2 * wa[...], axis=1, keepdims=True) + bav[...])
    mij = att * m2
    c = jnp.tanh(_dot16(mij, wc1[...]) + bc1[...])
    s = jnp.tanh(jnp.sum(c * wc2[...], axis=1, keepdims=True))
    m_out[...] = mij
    ct = d * s
    one = jnp.ones((ct.shape[0], 1), _f32)
    ct_out[...] = jnp.concatenate(
        [ct[:, :3], one, jnp.zeros((ct.shape[0], 12), _f32)], axis=1)


def _edge_mlp(g1, g2, c1, c2, wl2, wm2, bm2, wa, bav, wc1, bc1, wc2):
    grid = (E // BE,)
    full = lambda i: (0, 0)
    return pl.pallas_call(
        _edge_body,
        grid=grid,
        in_specs=[
            pl.BlockSpec((BE, MSG), lambda i: (i, 0)),
            pl.BlockSpec((BE, MSG), lambda i: (i, 0)),
            pl.BlockSpec((BE, 8), lambda i: (i, 0)),
            pl.BlockSpec((BE, 8), lambda i: (i, 0)),
            pl.BlockSpec((1, MSG), full),
            pl.BlockSpec((MSG, MSG), full),
            pl.BlockSpec((1, MSG), full),
            pl.BlockSpec((1, MSG), full),
            pl.BlockSpec((1, 1), full),
            pl.BlockSpec((MSG, MSG), full),
            pl.BlockSpec((1, MSG), full),
            pl.BlockSpec((1, MSG), full),
        ],
        out_specs=[
            pl.BlockSpec((BE, MSG), lambda i: (i, 0)),
            pl.BlockSpec((BE, 16), lambda i: (i, 0)),
        ],
        out_shape=[
            jax.ShapeDtypeStruct((E, MSG), _f32),
            jax.ShapeDtypeStruct((E, 16), _f32),
        ],
    )(g1, g2, c1, c2, wl2, wm2, bm2, wa, bav, wc1, bc1, wc2)


def _sc_scatter(m, ct, dst):
    @functools.partial(
        pl.kernel,
        out_type=(
            jax.ShapeDtypeStruct((NC * N, MSG), _f32),
            jax.ShapeDtypeStruct((NC * N, 16), _f32),
        ),
        mesh=_mesh(),
        scratch_types=[
            pltpu.VMEM((CH,), jnp.int32),
            pltpu.VMEM((CH,), jnp.int32),
            pltpu.VMEM((CH, MSG), _f32),
            pltpu.VMEM((CH, MSG), _f32),
            pltpu.VMEM((CH, 16), _f32),
            pltpu.VMEM((CH, 16), _f32),
            pltpu.VMEM((ZRS, MSG), _f32),
            pltpu.VMEM((ZRS, 16), _f32),
            pltpu.VMEM_SHARED((N, MSG), _f32),
            pltpu.VMEM_SHARED((N, 16), _f32),
            pltpu.SemaphoreType.DMA,
            pltpu.SemaphoreType.DMA,
        ],
        compiler_params=pltpu.CompilerParams(use_tc_tiling_on_sc=False),
    )
    def sk(m_h, ct_h, dst_h, pm_h, pct_h,
           idx0, idx1, rows0, rows1, crows0, crows1, zb, zbc, acc_m, acc_ct,
           seml0, seml1):
        cid = lax.axis_index("c")
        sid = lax.axis_index("s")
        zv = jnp.zeros((16,), _f32)
        idx = (idx0, idx1)
        rows = (rows0, rows1)
        crows = (crows0, crows1)
        seml = (seml0, seml1)
        sems = (seml0, seml1)
        base = (sid * NC + cid) * EPW

        def fire_loads(j, b):
            off = base + j * CH
            pltpu.async_copy(dst_h.at[pl.ds(off, CH)], idx[b], seml[b])
            pltpu.async_copy(m_h.at[pl.ds(off, CH)], rows[b], seml[b])
            pltpu.async_copy(ct_h.at[pl.ds(off, CH)], crows[b], seml[b])

        def drain_loads(b):
            pltpu.make_async_copy(dst_h.at[pl.ds(0, CH)], idx[b], seml[b]).wait()
            pltpu.make_async_copy(m_h.at[pl.ds(0, CH)], rows[b], seml[b]).wait()
            pltpu.make_async_copy(ct_h.at[pl.ds(0, CH)], crows[b], seml[b]).wait()

        def fire_scat(b):
            pltpu.async_copy(rows[b], acc_m.at[idx[b]], sems[b], add=True)
            pltpu.async_copy(crows[b], acc_ct.at[idx[b]], sems[b], add=True)

        def drain_scat(b):
            pltpu.make_async_copy(rows[b], acc_m.at[idx[b]], sems[b]).wait()
            pltpu.make_async_copy(crows[b], acc_ct.at[idx[b]], sems[b]).wait()

        fire_loads(0, 0)
        fire_loads(1, 1)

        def zrow(t, carry):
            zb[t // 8, pl.ds((t % 8) * 16, 16)] = zv
            return carry

        lax.fori_loop(0, ZRS * 8, zrow, 0)

        def zrow2(i, carry):
            zbc[i, :] = zv
            return carry

        lax.fori_loop(0, ZRS, zrow2, 0)

        def zcopy(t, carry):
            ds = pl.ds(sid * STRIPE + t * ZRS, ZRS)
            pltpu.sync_copy(zb, acc_m.at[ds])
            pltpu.sync_copy(zbc, acc_ct.at[ds])
            return carry

        lax.fori_loop(0, STRIPE // ZRS, zcopy, 0)
        plsc.subcore_barrier()

        def body(jj, carry):
            for b in range(2):
                j = 2 * jj + b
                drain_loads(b)
                fire_scat(b)
                drain_scat(b)

                @pl.when(j + 2 < NIT2)
                def _():
                    fire_loads(j + 2, b)
            return carry

        lax.fori_loop(0, NIT2 // 2, body, 0)
        drain_loads(0)
        fire_scat(0)
        drain_scat(0)
        plsc.subcore_barrier()

        out_ds = pl.ds(cid * N + sid * STRIPE, STRIPE)
        pltpu.sync_copy(acc_m.at[pl.ds(sid * STRIPE, STRIPE)], pm_h.at[out_ds])
        pltpu.sync_copy(acc_ct.at[pl.ds(sid * STRIPE, STRIPE)], pct_h.at[out_ds])

    return sk(m, ct, dst)


def _node_body(h, c8, pm0, pm1, pct0, pct1, wh1a, wh1b, b1, wh2, b2,
               ho, co):
    mi = pm0[...] + pm1[...]
    ctv = pct0[...] + pct1[...]
    cnt = ctv[:, 3:4]
    co[...] = c8[...] + ctv[:, :8] / cnt
    hv = h[...]
    hm = jnp.tanh(_dot(hv, wh1a[...]) + _dot(mi, wh1b[...]) + b1[...])
    ho[...] = hv + _dot(hm, wh2[...]) + b2[...]


def _node(hidden, c8, pm, pct, wh1a, wh1b, b1, wh2, b2):
    grid = (N // BN,)
    full = lambda i: (0, 0)
    nb = N // BN
    return pl.pallas_call(
        _node_body,
        grid=grid,
        in_specs=[
            pl.BlockSpec((BN, HID), lambda i: (i, 0)),
            pl.BlockSpec((BN, 8), lambda i: (i, 0)),
            pl.BlockSpec((BN, MSG), lambda i: (i, 0)),
            pl.BlockSpec((BN, MSG), lambda i: (i + nb, 0)),
            pl.BlockSpec((BN, 16), lambda i: (i, 0)),
            pl.BlockSpec((BN, 16), lambda i: (i + nb, 0)),
            pl.BlockSpec((HID, MSG), full),
            pl.BlockSpec((MSG, MSG), full),
            pl.BlockSpec((1, MSG), full),
            pl.BlockSpec((MSG, HID), full),
            pl.BlockSpec((1, HID), full),
        ],
        out_specs=[
            pl.BlockSpec((BN, HID), lambda i: (i, 0)),
            pl.BlockSpec((BN, 8), lambda i: (i, 0)),
        ],
        out_shape=[
            jax.ShapeDtypeStruct((N, HID), _f32),
            jax.ShapeDtypeStruct((N, 8), _f32),
        ],
    )(hidden, c8, pm, pm, pct, pct, wh1a, wh1b, b1, wh2, b2)


def kernel(coords, hidden, edges, Wm1, bm1, Wm2, bm2, Wc1, bc1, Wc2,
           Wa, ba, Wh1, bh1, Wh2, bh2):
    src = edges[0].astype(jnp.int32)
    dst = edges[1].astype(jnp.int32)
    c8 = jnp.pad(coords, ((0, 0), (0, 5)))

    w1a = Wm1[:HID]
    w1b = Wm1[HID:2 * HID]
    wl2 = Wm1[2 * HID:]
    t1, t2 = _precompute(hidden, w1a, w1b, bm1.reshape(1, MSG))

    g1, g2 = _sc_gather_wide(src, dst, t1, t2)
    ce1, ce2 = _sc_gather_coords(src, dst, c8)

    m, ct = _edge_mlp(
        g1, g2, ce1, ce2, wl2, Wm2, bm2.reshape(1, MSG),
        Wa.reshape(1, MSG), ba.reshape(1, 1), Wc1, bc1.reshape(1, MSG),
        Wc2.reshape(1, MSG))

    pm, pct = _sc_scatter(m, ct, dst)

    ho, co8 = _node(
        hidden, c8, pm, pct,
        Wh1[:HID], Wh1[HID:], bh1.reshape(1, MSG), Wh2,
        bh2.reshape(1, HID))

    return (co8[:, :3], ho)


# speedup vs baseline: 1.3958x; 1.0871x over previous
import functools

import jax
import jax.numpy as jnp
from jax import lax
from jax.experimental import pallas as pl
from jax.experimental.pallas import tpu as pltpu
from jax.experimental.pallas import tpu_sc as plsc

N = 10000
E = 320000
HID = 128
MSG = 128

NC = 2
NS = 16
NW = NC * NS
EPW = E // NW
CH = 80
SUB = 5
S = CH * SUB
NIT1 = EPW // S
NIT2 = EPW // CH
STRIPE = N // NS
ZR = STRIPE // 5
ZRS = 25

BE = 6400
BN = 1000

_P = jax.lax.Precision.HIGHEST
_f32 = jnp.float32


def _mesh():
    return plsc.VectorSubcoreMesh(
        core_axis_name="c", subcore_axis_name="s", num_cores=NC, num_subcores=NS)


def _dot(a, b):
    return jax.lax.dot(a, b, precision=_P, preferred_element_type=_f32)


def _dot16(a, b):
    return jax.lax.dot(a.astype(jnp.bfloat16), b.astype(jnp.bfloat16),
                       preferred_element_type=_f32)


def _pre_body(h, w1a, w1b, b1, t1, t2):
    hv = h[...]
    t1[...] = _dot(hv, w1a[...]) + b1[...]
    t2[...] = _dot(hv, w1b[...])


def _precompute(hidden, w1a, w1b, b1):
    grid = (N // BN,)
    return pl.pallas_call(
        _pre_body,
        grid=grid,
        in_specs=[
            pl.BlockSpec((BN, HID), lambda i: (i, 0)),
            pl.BlockSpec((HID, MSG), lambda i: (0, 0)),
            pl.BlockSpec((HID, MSG), lambda i: (0, 0)),
            pl.BlockSpec((1, MSG), lambda i: (0, 0)),
        ],
        out_specs=[
            pl.BlockSpec((BN, MSG), lambda i: (i, 0)),
            pl.BlockSpec((BN, MSG), lambda i: (i, 0)),
        ],
        out_shape=[
            jax.ShapeDtypeStruct((N, MSG), _f32),
            jax.ShapeDtypeStruct((N, MSG), _f32),
        ],
    )(hidden, w1a, w1b, b1)


SG = 200
GSUB = (128, 72)
NSTEP = EPW // SG


def _sc_gather_wide(src, dst, t1, t2):
    @functools.partial(
        pl.kernel,
        out_type=(
            jax.ShapeDtypeStruct((E, MSG), _f32),
            jax.ShapeDtypeStruct((E, MSG), _f32),
        ),
        mesh=_mesh(),
        scratch_types=[
            pltpu.VMEM((SG,), jnp.int32),
            pltpu.VMEM((SG,), jnp.int32),
            pltpu.VMEM((SG,), jnp.int32),
            pltpu.VMEM((SG,), jnp.int32),
            pltpu.VMEM((SG, MSG), _f32),
            pltpu.VMEM((SG, MSG), _f32),
            pltpu.VMEM((SG, MSG), _f32),
            pltpu.VMEM((SG, MSG), _f32),
            pltpu.SemaphoreType.DMA,
            pltpu.SemaphoreType.DMA,
            pltpu.SemaphoreType.DMA,
            pltpu.SemaphoreType.DMA,
            pltpu.SemaphoreType.DMA,
            pltpu.SemaphoreType.DMA,
        ],
    )
    def gk(src_h, dst_h, t1_h, t2_h, g1_h, g2_h,
           idxs0, idxs1, idxd0, idxd1, ra0, ra1, rb0, rb1,
           semi0, semi1, semg0, semg1, semo0, semo1):
        wid = lax.axis_index("s") * NC + lax.axis_index("c")
        base = wid * EPW
        idxs = (idxs0, idxs1)
        idxd = (idxd0, idxd1)
        ra = (ra0, ra1)
        rb = (rb0, rb1)
        semi = (semi0, semi1)
        semg = (semg0, semg1)
        semo = (semo0, semo1)

        def fire_idx(j, b):
            off = base + j * SG
            pltpu.async_copy(src_h.at[pl.ds(off, SG)], idxs[b], semi[b])
            pltpu.async_copy(dst_h.at[pl.ds(off, SG)], idxd[b], semi[b])

        def drain_idx(b):
            pltpu.make_async_copy(src_h.at[pl.ds(0, SG)], idxs[b], semi[b]).wait()
            pltpu.make_async_copy(dst_h.at[pl.ds(0, SG)], idxd[b], semi[b]).wait()

        def fire_gathers(b):
            o = 0
            for c in GSUB:
                sl = pl.ds(o, c)
                pltpu.async_copy(t1_h.at[idxs[b].at[sl]], ra[b].at[sl], semg[b])
                pltpu.async_copy(t2_h.at[idxd[b].at[sl]], rb[b].at[sl], semg[b])
                o += c

        def drain_gathers(b):
            o = 0
            for c in GSUB:
                sl = pl.ds(o, c)
                pltpu.make_async_copy(
                    t1_h.at[idxs[b].at[sl]], ra[b].at[sl], semg[b]).wait()
                pltpu.make_async_copy(
                    t2_h.at[idxd[b].at[sl]], rb[b].at[sl], semg[b]).wait()
                o += c

        def fire_stores(j, b):
            off = base + j * SG
            pltpu.async_copy(ra[b], g1_h.at[pl.ds(off, SG)], semo[b])
            pltpu.async_copy(rb[b], g2_h.at[pl.ds(off, SG)], semo[b])

        def drain_stores(b):
            pltpu.make_async_copy(ra[b], g1_h.at[pl.ds(0, SG)], semo[b]).wait()
            pltpu.make_async_copy(rb[b], g2_h.at[pl.ds(0, SG)], semo[b]).wait()

        fire_idx(0, 0)
        fire_idx(1, 1)

        def body(jj, carry):
            for b in range(2):
                j = 2 * jj + b

                @pl.when(jj > 0)
                def _():
                    drain_stores(b)

                drain_idx(b)
                fire_gathers(b)
                drain_gathers(b)

                @pl.when(j + 2 < NSTEP)
                def _():
                    fire_idx(j + 2, b)

                fire_stores(j, b)
            return carry

        lax.fori_loop(0, NSTEP // 2, body, 0)
        drain_stores(0)
        drain_stores(1)

    return gk(src, dst, t1, t2)


def _sc_gather_coords(src, dst, c8):
    @functools.partial(
        pl.kernel,
        out_type=jax.ShapeDtypeStruct((E, 16), _f32),
        mesh=_mesh(),
        scratch_types=[
            pltpu.VMEM((S,), jnp.int32),
            pltpu.VMEM((S,), jnp.int32),
            pltpu.VMEM((S, 8), _f32),
            pltpu.VMEM((S, 8), _f32),
            pltpu.SemaphoreType.DMA,
            pltpu.SemaphoreType.DMA,
        ],
        compiler_params=pltpu.CompilerParams(use_tc_tiling_on_sc=False),
    )
    def gk(src_h, dst_h, c8_h, cc_h, idxs, idxd, ca, cb, sem1, sem2):
        wid = lax.axis_index("s") * NC + lax.axis_index("c")
        base = wid * EPW

        def body(j, carry):
            off = base + j * S
            cp1 = pltpu.async_copy(src_h.at[pl.ds(off, S)], idxs, sem1)
            cp2 = pltpu.async_copy(dst_h.at[pl.ds(off, S)], idxd, sem1)
            cp1.wait()
            cp2.wait()
            descs = []
            for k in range(SUB):
                sl = pl.ds(k * CH, CH)
                descs.append(pltpu.async_copy(c8_h.at[idxs.at[sl]], ca.at[sl], sem2))
                descs.append(pltpu.async_copy(c8_h.at[idxd.at[sl]], cb.at[sl], sem2))
            for d in descs:
                d.wait()
            pltpu.sync_copy(ca, cc_h.at[pl.ds(off, S), pl.ds(0, 8)])
            pltpu.sync_copy(cb, cc_h.at[pl.ds(off, S), pl.ds(8, 8)])
            return carry

        lax.fori_loop(0, NIT1, body, 0)

    return gk(src, dst, c8)


def _edge_body(g1, g2, cc, wl2, wm2, bm2, wa, bav, wc1, bc1, wc2,
               m_out, ct_out):
    ccv = cc[...]
    d = ccv[:, :8] - ccv[:, 8:]
    l2 = jnp.sqrt(jnp.sum(d * d, axis=1, keepdims=True))
    pre = g1[...] + g2[...] + l2 * wl2[...]
    m1 = jnp.tanh(pre)
    m2 = jnp.tanh(_dot16(m1, wm2[...]) + bm2[...])
    att = jax.nn.sigmoid(
        jnp.sum(m2 * wa[...], axis=1, keepdims=True) + bav[...])
    mij = att * m2
    c = jnp.tanh(_dot16(mij, wc1[...]) + bc1[...])
    s = jnp.tanh(jnp.sum(c * wc2[...], axis=1, keepdims=True))
    m_out[...] = mij
    ct = d * s
    one = jnp.ones((ct.shape[0], 1), _f32)
    ct_out[...] = jnp.concatenate(
        [ct[:, :3], one, jnp.zeros((ct.shape[0], 12), _f32)], axis=1)


def _edge_mlp(g1, g2, cc, wl2, wm2, bm2, wa, bav, wc1, bc1, wc2):
    grid = (E // BE,)
    full = lambda i: (0, 0)
    return pl.pallas_call(
        _edge_body,
        grid=grid,
        in_specs=[
            pl.BlockSpec((BE, MSG), lambda i: (i, 0)),
            pl.BlockSpec((BE, MSG), lambda i: (i, 0)),
            pl.BlockSpec((BE, 16), lambda i: (i, 0)),
            pl.BlockSpec((1, MSG), full),
            pl.BlockSpec((MSG, MSG), full),
            pl.BlockSpec((1, MSG), full),
            pl.BlockSpec((1, MSG), full),
            pl.BlockSpec((1, 1), full),
            pl.BlockSpec((MSG, MSG), full),
            pl.BlockSpec((1, MSG), full),
            pl.BlockSpec((1, MSG), full),
        ],
        out_specs=[
            pl.BlockSpec((BE, MSG), lambda i: (i, 0)),
            pl.BlockSpec((BE, 16), lambda i: (i, 0)),
        ],
        out_shape=[
            jax.ShapeDtypeStruct((E, MSG), _f32),
            jax.ShapeDtypeStruct((E, 16), _f32),
        ],
    )(g1, g2, cc, wl2, wm2, bm2, wa, bav, wc1, bc1, wc2)


def _sc_scatter(m, ct, dst):
    @functools.partial(
        pl.kernel,
        out_type=(
            jax.ShapeDtypeStruct((NC * N, MSG), _f32),
            jax.ShapeDtypeStruct((NC * N, 16), _f32),
        ),
        mesh=_mesh(),
        scratch_types=[
            pltpu.VMEM((CH,), jnp.int32),
            pltpu.VMEM((CH,), jnp.int32),
            pltpu.VMEM((CH, MSG), _f32),
            pltpu.VMEM((CH, MSG), _f32),
            pltpu.VMEM((CH, 16), _f32),
            pltpu.VMEM((CH, 16), _f32),
            pltpu.VMEM((ZRS, MSG), _f32),
            pltpu.VMEM((ZRS, 16), _f32),
            pltpu.VMEM_SHARED((N, MSG), _f32),
            pltpu.VMEM_SHARED((N, 16), _f32),
            pltpu.SemaphoreType.DMA,
            pltpu.SemaphoreType.DMA,
        ],
        compiler_params=pltpu.CompilerParams(use_tc_tiling_on_sc=False),
    )
    def sk(m_h, ct_h, dst_h, pm_h, pct_h,
           idx0, idx1, rows0, rows1, crows0, crows1, zb, zbc, acc_m, acc_ct,
           seml0, seml1):
        cid = lax.axis_index("c")
        sid = lax.axis_index("s")
        zv = jnp.zeros((16,), _f32)
        idx = (idx0, idx1)
        rows = (rows0, rows1)
        crows = (crows0, crows1)
        seml = (seml0, seml1)
        sems = (seml0, seml1)
        base = (sid * NC + cid) * EPW

        def fire_loads(j, b):
            off = base + j * CH
            pltpu.async_copy(dst_h.at[pl.ds(off, CH)], idx[b], seml[b])
            pltpu.async_copy(m_h.at[pl.ds(off, CH)], rows[b], seml[b])
            pltpu.async_copy(ct_h.at[pl.ds(off, CH)], crows[b], seml[b])

        def drain_loads(b):
            pltpu.make_async_copy(dst_h.at[pl.ds(0, CH)], idx[b], seml[b]).wait()
            pltpu.make_async_copy(m_h.at[pl.ds(0, CH)], rows[b], seml[b]).wait()
            pltpu.make_async_copy(ct_h.at[pl.ds(0, CH)], crows[b], seml[b]).wait()

        def fire_scat(b):
            pltpu.async_copy(rows[b], acc_m.at[idx[b]], sems[b], add=True)
            pltpu.async_copy(crows[b], acc_ct.at[idx[b]], sems[b], add=True)

        def drain_scat(b):
            pltpu.make_async_copy(rows[b], acc_m.at[idx[b]], sems[b]).wait()
            pltpu.make_async_copy(crows[b], acc_ct.at[idx[b]], sems[b]).wait()

        fire_loads(0, 0)
        fire_loads(1, 1)

        def zrow(t, carry):
            zb[t // 8, pl.ds((t % 8) * 16, 16)] = zv
            return carry

        lax.fori_loop(0, ZRS * 8, zrow, 0)

        def zrow2(i, carry):
            zbc[i, :] = zv
            return carry

        lax.fori_loop(0, ZRS, zrow2, 0)

        def zcopy(t, carry):
            ds = pl.ds(sid * STRIPE + t * ZRS, ZRS)
            pltpu.sync_copy(zb, acc_m.at[ds])
            pltpu.sync_copy(zbc, acc_ct.at[ds])
            return carry

        lax.fori_loop(0, STRIPE // ZRS, zcopy, 0)
        plsc.subcore_barrier()

        def body(jj, carry):
            for b in range(2):
                j = 2 * jj + b
                drain_loads(b)
                fire_scat(b)
                drain_scat(b)

                @pl.when(j + 2 < NIT2)
                def _():
                    fire_loads(j + 2, b)
            return carry

        lax.fori_loop(0, NIT2 // 2, body, 0)
        drain_loads(0)
        fire_scat(0)
        drain_scat(0)
        plsc.subcore_barrier()

        out_ds = pl.ds(cid * N + sid * STRIPE, STRIPE)
        pltpu.sync_copy(acc_m.at[pl.ds(sid * STRIPE, STRIPE)], pm_h.at[out_ds])
        pltpu.sync_copy(acc_ct.at[pl.ds(sid * STRIPE, STRIPE)], pct_h.at[out_ds])

    return sk(m, ct, dst)


def _node_body(h, c8, pm0, pm1, pct0, pct1, wh1a, wh1b, b1, wh2, b2,
               ho, co):
    mi = pm0[...] + pm1[...]
    ctv = pct0[...] + pct1[...]
    cnt = ctv[:, 3:4]
    co[...] = c8[...] + ctv[:, :8] / cnt
    hv = h[...]
    hm = jnp.tanh(_dot(hv, wh1a[...]) + _dot(mi, wh1b[...]) + b1[...])
    ho[...] = hv + _dot(hm, wh2[...]) + b2[...]


def _node(hidden, c8, pm, pct, wh1a, wh1b, b1, wh2, b2):
    grid = (N // BN,)
    full = lambda i: (0, 0)
    nb = N // BN
    return pl.pallas_call(
        _node_body,
        grid=grid,
        in_specs=[
            pl.BlockSpec((BN, HID), lambda i: (i, 0)),
            pl.BlockSpec((BN, 8), lambda i: (i, 0)),
            pl.BlockSpec((BN, MSG), lambda i: (i, 0)),
            pl.BlockSpec((BN, MSG), lambda i: (i + nb, 0)),
            pl.BlockSpec((BN, 16), lambda i: (i, 0)),
            pl.BlockSpec((BN, 16), lambda i: (i + nb, 0)),
            pl.BlockSpec((HID, MSG), full),
            pl.BlockSpec((MSG, MSG), full),
            pl.BlockSpec((1, MSG), full),
            pl.BlockSpec((MSG, HID), full),
            pl.BlockSpec((1, HID), full),
        ],
        out_specs=[
            pl.BlockSpec((BN, HID), lambda i: (i, 0)),
            pl.BlockSpec((BN, 8), lambda i: (i, 0)),
        ],
        out_shape=[
            jax.ShapeDtypeStruct((N, HID), _f32),
            jax.ShapeDtypeStruct((N, 8), _f32),
        ],
    )(hidden, c8, pm, pm, pct, pct, wh1a, wh1b, b1, wh2, b2)


def kernel(coords, hidden, edges, Wm1, bm1, Wm2, bm2, Wc1, bc1, Wc2,
           Wa, ba, Wh1, bh1, Wh2, bh2):
    src = edges[0].astype(jnp.int32)
    dst = edges[1].astype(jnp.int32)
    c8 = jnp.pad(coords, ((0, 0), (0, 5)))

    w1a = Wm1[:HID]
    w1b = Wm1[HID:2 * HID]
    wl2 = Wm1[2 * HID:]
    t1, t2 = _precompute(hidden, w1a, w1b, bm1.reshape(1, MSG))

    g1, g2 = _sc_gather_wide(src, dst, t1, t2)
    cc = _sc_gather_coords(src, dst, c8)

    m, ct = _edge_mlp(
        g1, g2, cc, wl2, Wm2, bm2.reshape(1, MSG),
        Wa.reshape(1, MSG), ba.reshape(1, 1), Wc1, bc1.reshape(1, MSG),
        Wc2.reshape(1, MSG))

    pm, pct = _sc_scatter(m, ct, dst)

    ho, co8 = _node(
        hidden, c8, pm, pct,
        Wh1[:HID], Wh1[HID:], bh1.reshape(1, MSG), Wh2,
        bh2.reshape(1, HID))

    return (co8[:, :3], ho)
